# baseline (speedup 1.0000x reference)
import functools
import jax
import jax.numpy as jnp
from jax import lax
from jax.experimental import pallas as pl
from jax.experimental.pallas import tpu as pltpu
from jax.experimental.pallas import tpu_sc as plsc

NC = 2
NS = 16
NW = NC * NS

BT = 128
SB = 4
L = 16


def _make_kernel(B, S, V, D):
    assert B == BT * NW and S % (2 * SB) == 0 and D % 8 == 0
    STEPS = S // SB
    DT = D // 8
    CC = BT // L

    mesh = plsc.VectorSubcoreMesh(core_axis_name="c", subcore_axis_name="s")

    @functools.partial(
        pl.kernel,
        mesh=mesh,
        out_type=jax.ShapeDtypeStruct((S, DT, NW, 8, BT), jnp.float32),
        scratch_types=[
            pltpu.VMEM((2, SB, BT), jnp.int32),
            pltpu.VMEM((2, SB, BT, D), jnp.float32),
            pltpu.VMEM((2, DT, SB, 8, BT), jnp.float32),
            pltpu.VMEM((2, SB, D, L), jnp.float32),
            pltpu.SemaphoreType.DMA,
            pltpu.SemaphoreType.DMA,
            pltpu.SemaphoreType.DMA,
            pltpu.SemaphoreType.DMA,
        ],
        compiler_params=pltpu.CompilerParams(use_tc_tiling_on_sc=False, needs_layout_passes=False),
    )
    def k(idx_hbm, table_hbm, pos_hbm, out_hbm, idx_v, rows_v, x_v, pos_v,
          gsem0, gsem1, ssem0, ssem1):
        wid = lax.axis_index("s") * NC + lax.axis_index("c")
        bbase = wid * BT
        gsems = (gsem0, gsem1)
        ssems = (ssem0, ssem1)

        def fire(g, b):
            s0 = g * SB
            pltpu.sync_copy(
                idx_hbm.at[pl.ds(s0, SB), pl.ds(bbase, BT)], idx_v.at[b]
            )
            pltpu.sync_copy(pos_hbm.at[pl.ds(s0, SB)], pos_v.at[b])
            for sl in range(SB):
                pltpu.async_copy(
                    table_hbm.at[idx_v.at[b, sl]], rows_v.at[b, sl], gsems[b]
                )

        def store_waits(g, b):
            for ti in range(DT):
                for sl in range(SB):
                    pltpu.make_async_copy(
                        x_v.at[b, ti, sl], out_hbm.at[g * SB + sl, ti, wid],
                        ssems[b],
                    ).wait()

        def process(g, b):
            for sl in range(SB):
                pltpu.make_async_copy(
                    table_hbm.at[idx_v.at[b, sl]], rows_v.at[b, sl], gsems[b]
                ).wait()

            @pl.when(g >= 2)
            def _():
                store_waits(g - 2, b)

            lanes = lax.iota(jnp.int32, L)

            def per_d(d, carry):
                ti = d // 8
                r = d - ti * 8
                dvec = jnp.full((L,), d, jnp.int32)
                for sl in range(SB):
                    pvec = pos_v[b, sl, d, :]
                    for cc in range(CC):
                        v = plsc.load_gather(
                            rows_v.at[b, sl], [lanes + cc * L, dvec]
                        )
                        x_v[b, ti, sl, r, pl.ds(cc * L, L)] = v + pvec
                return carry

            lax.fori_loop(0, D, per_d, 0)

            for ti in range(DT):
                for sl in range(SB):
                    pltpu.async_copy(
                        x_v.at[b, ti, sl], out_hbm.at[g * SB + sl, ti, wid],
                        ssems[b],
                    )

        fire(0, 0)

        def body(gg, carry):
            g0 = 2 * gg
            fire(g0 + 1, 1)
            process(g0, 0)

            @pl.when(g0 + 2 < STEPS)
            def _():
                fire(g0 + 2, 0)

            process(g0 + 1, 1)
            return carry

        lax.fori_loop(0, STEPS // 2, body, 0)
        store_waits(STEPS - 2, 0)
        store_waits(STEPS - 1, 1)

    return k


def kernel(inputs, word_table, pos_embed):
    B, S = inputs.shape
    V, D = word_table.shape
    idx_t = inputs.astype(jnp.int32).T
    pos_rep = jnp.broadcast_to(pos_embed[:, :, None], (S, D, L))
    x = _make_kernel(B, S, V, D)(idx_t, word_table, pos_rep)
    return x.transpose(2, 4, 0, 1, 3).reshape(B, S, D)


# speedup vs baseline: 1.0042x; 1.0042x over previous
import functools
import jax
import jax.numpy as jnp
from jax import lax
from jax.experimental import pallas as pl
from jax.experimental.pallas import tpu as pltpu
from jax.experimental.pallas import tpu_sc as plsc

NC = 2
NS = 16
NW = NC * NS

BT = 128
SB = 4
L = 16


def _make_kernel(B, S, V, D):
    assert B == BT * NW and S % (2 * SB) == 0 and D % 8 == 0
    STEPS = S // SB
    DT = D // 8
    CC = BT // L

    mesh = plsc.VectorSubcoreMesh(core_axis_name="c", subcore_axis_name="s")

    @functools.partial(
        pl.kernel,
        mesh=mesh,
        out_type=jax.ShapeDtypeStruct((S, DT, NW, 8, BT), jnp.float32),
        scratch_types=[
            pltpu.VMEM((2, SB, BT), jnp.int32),
            pltpu.VMEM((2, SB, BT, D), jnp.float32),
            pltpu.VMEM((2, SB, DT, 8, BT), jnp.float32),
            pltpu.SemaphoreType.DMA,
            pltpu.SemaphoreType.DMA,
            pltpu.SemaphoreType.DMA,
            pltpu.SemaphoreType.DMA,
        ],
        compiler_params=pltpu.CompilerParams(use_tc_tiling_on_sc=False, needs_layout_passes=False),
    )
    def k(idx_hbm, table_hbm, posx_hbm, out_hbm, idx_v, rows_v, x_v,
          gsem0, gsem1, ssem0, ssem1):
        wid = lax.axis_index("s") * NC + lax.axis_index("c")
        bbase = wid * BT
        gsems = (gsem0, gsem1)
        ssems = (ssem0, ssem1)

        def fire(g, b):
            s0 = g * SB
            pltpu.sync_copy(
                idx_hbm.at[pl.ds(s0, SB), pl.ds(bbase, BT)], idx_v.at[b]
            )
            for sl in range(SB):
                pltpu.async_copy(
                    table_hbm.at[idx_v.at[b, sl]], rows_v.at[b, sl], gsems[b]
                )

        def store_waits(g, b):
            for sl in range(SB):
                for ti in range(DT):
                    pltpu.make_async_copy(
                        x_v.at[b, sl, ti], out_hbm.at[g * SB + sl, ti, wid],
                        ssems[b],
                    ).wait()

        def process(g, b):
            @pl.when(g >= 2)
            def _():
                store_waits(g - 2, b)

            pltpu.sync_copy(posx_hbm.at[pl.ds(g * SB, SB)], x_v.at[b])

            for sl in range(SB):
                pltpu.make_async_copy(
                    table_hbm.at[idx_v.at[b, sl]], rows_v.at[b, sl], gsems[b]
                ).wait()

            lanes = lax.iota(jnp.int32, L)

            def per_cc(cc, carry):
                rvec = lanes + cc * L
                for sl in range(SB):
                    for d in range(D):
                        v = plsc.load_gather(
                            rows_v.at[b, sl], [rvec, jnp.full((L,), d, jnp.int32)]
                        )
                        plsc.addupdate(
                            x_v.at[b, sl, d // 8, d % 8, pl.ds(cc * L, L)], v
                        )
                return carry

            lax.fori_loop(0, CC, per_cc, 0)

            for sl in range(SB):
                for ti in range(DT):
                    pltpu.async_copy(
                        x_v.at[b, sl, ti], out_hbm.at[g * SB + sl, ti, wid],
                        ssems[b],
                    )

        fire(0, 0)

        def body(gg, carry):
            g0 = 2 * gg
            fire(g0 + 1, 1)
            process(g0, 0)

            @pl.when(g0 + 2 < STEPS)
            def _():
                fire(g0 + 2, 0)

            process(g0 + 1, 1)
            return carry

        lax.fori_loop(0, STEPS // 2, body, 0)
        store_waits(STEPS - 2, 0)
        store_waits(STEPS - 1, 1)

    return k


def kernel(inputs, word_table, pos_embed):
    B, S = inputs.shape
    V, D = word_table.shape
    idx_t = inputs.astype(jnp.int32).T
    pos_x = jnp.broadcast_to(
        pos_embed.reshape(S, D // 8, 8)[:, :, :, None], (S, D // 8, 8, BT)
    )
    x = _make_kernel(B, S, V, D)(idx_t, word_table, pos_x)
    return x.transpose(2, 4, 0, 1, 3).reshape(B, S, D)


# speedup vs baseline: 1.3465x; 1.3409x over previous
import functools
import jax
import jax.numpy as jnp
from jax import lax
from jax.experimental import pallas as pl
from jax.experimental.pallas import tpu as pltpu
from jax.experimental.pallas import tpu_sc as plsc

NC = 2
NS = 16
NW = NC * NS

BT = 128
SB = 4
L = 16


def _make_kernel(B, S, V, D):
    assert B == BT * NW and S % (2 * SB) == 0 and D % 8 == 0
    STEPS = S // SB
    DT = D // 8
    CC = BT // L

    mesh = plsc.VectorSubcoreMesh(core_axis_name="c", subcore_axis_name="s")

    @functools.partial(
        pl.kernel,
        mesh=mesh,
        out_type=jax.ShapeDtypeStruct((S, DT, NW, 8, BT), jnp.float32),
        scratch_types=[
            pltpu.VMEM((2, SB, BT), jnp.int32),
            pltpu.VMEM((2, SB, BT, D), jnp.float32),
            pltpu.VMEM((2, SB, DT, 8, BT), jnp.float32),
            pltpu.SemaphoreType.DMA,
            pltpu.SemaphoreType.DMA,
            pltpu.SemaphoreType.DMA,
            pltpu.SemaphoreType.DMA,
        ],
        compiler_params=pltpu.CompilerParams(use_tc_tiling_on_sc=False, needs_layout_passes=False),
    )
    def k(idx_hbm, table_hbm, posx_hbm, out_hbm, idx_v, rows_v, x_v,
          gsem0, gsem1, ssem0, ssem1):
        wid = lax.axis_index("s") * NC + lax.axis_index("c")
        bbase = wid * BT
        gsems = (gsem0, gsem1)
        ssems = (ssem0, ssem1)

        def fire(g, b):
            s0 = g * SB
            pltpu.sync_copy(
                idx_hbm.at[pl.ds(s0, SB), pl.ds(bbase, BT)], idx_v.at[b]
            )
            for sl in range(SB):
                pltpu.async_copy(
                    table_hbm.at[idx_v.at[b, sl]], rows_v.at[b, sl], gsems[b]
                )

        def store_waits(g, b):
            for sl in range(SB):
                for ti in range(DT):
                    pltpu.make_async_copy(
                        x_v.at[b, sl, ti], out_hbm.at[g * SB + sl, ti, wid],
                        ssems[b],
                    ).wait()

        def process(g, b):
            @pl.when(g >= 2)
            def _():
                store_waits(g - 2, b)

            pltpu.sync_copy(posx_hbm.at[pl.ds(g * SB, SB)], x_v.at[b])

            for sl in range(SB):
                pltpu.make_async_copy(
                    table_hbm.at[idx_v.at[b, sl]], rows_v.at[b, sl], gsems[b]
                ).wait()

            lanes = lax.iota(jnp.int32, L)

            @plsc.parallel_loop(0, CC * SB, 1, unroll=4)
            def _(i):
                cc = i // SB
                sl = i - cc * SB
                rvec = lanes + cc * L
                for d in range(D):
                    v = plsc.load_gather(
                        rows_v.at[b, sl], [rvec, jnp.full((L,), d, jnp.int32)]
                    )
                    plsc.addupdate(
                        x_v.at[b, sl, d // 8, d % 8, pl.ds(cc * L, L)], v
                    )

            for sl in range(SB):
                for ti in range(DT):
                    pltpu.async_copy(
                        x_v.at[b, sl, ti], out_hbm.at[g * SB + sl, ti, wid],
                        ssems[b],
                    )

        fire(0, 0)

        def body(gg, carry):
            g0 = 2 * gg
            fire(g0 + 1, 1)
            process(g0, 0)

            @pl.when(g0 + 2 < STEPS)
            def _():
                fire(g0 + 2, 0)

            process(g0 + 1, 1)
            return carry

        lax.fori_loop(0, STEPS // 2, body, 0)
        store_waits(STEPS - 2, 0)
        store_waits(STEPS - 1, 1)

    return k


def kernel(inputs, word_table, pos_embed):
    B, S = inputs.shape
    V, D = word_table.shape
    idx_t = inputs.astype(jnp.int32).T
    pos_x = jnp.broadcast_to(
        pos_embed.reshape(S, D // 8, 8)[:, :, :, None], (S, D // 8, 8, BT)
    )
    x = _make_kernel(B, S, V, D)(idx_t, word_table, pos_x)
    return x.transpose(2, 4, 0, 1, 3).reshape(B, S, D)


# speedup vs baseline: 2.2484x; 1.6698x over previous
import functools
import jax
import jax.numpy as jnp
from jax import lax
from jax.experimental import pallas as pl
from jax.experimental.pallas import tpu as pltpu
from jax.experimental.pallas import tpu_sc as plsc

NC = 2
NS = 16
NW = NC * NS

BT = 128
SB = 4
L = 16
PAD = 33


def _make_kernel(B, S, V, D):
    assert B == BT * NW and S % (2 * SB) == 0 and D % 8 == 0
    STEPS = S // SB
    DT = D // 8
    CC = BT // L

    mesh = plsc.VectorSubcoreMesh(core_axis_name="c", subcore_axis_name="s")

    @functools.partial(
        pl.kernel,
        mesh=mesh,
        out_type=jax.ShapeDtypeStruct((S, DT, NW, 8, BT), jnp.float32),
        scratch_types=[
            pltpu.VMEM((2, SB, BT), jnp.int32),
            pltpu.VMEM((2, SB, BT, PAD), jnp.float32),
            pltpu.VMEM((2, SB, DT, 8, BT), jnp.float32),
            pltpu.SemaphoreType.DMA,
            pltpu.SemaphoreType.DMA,
            pltpu.SemaphoreType.DMA,
            pltpu.SemaphoreType.DMA,
        ],
        compiler_params=pltpu.CompilerParams(use_tc_tiling_on_sc=False, needs_layout_passes=False),
    )
    def k(idx_hbm, table_hbm, posx_hbm, out_hbm, idx_v, rows_v, x_v,
          gsem0, gsem1, ssem0, ssem1):
        wid = lax.axis_index("s") * NC + lax.axis_index("c")
        bbase = wid * BT
        gsems = (gsem0, gsem1)
        ssems = (ssem0, ssem1)

        def fire(g, b):
            s0 = g * SB
            pltpu.sync_copy(
                idx_hbm.at[pl.ds(s0, SB), pl.ds(bbase, BT)], idx_v.at[b]
            )
            for sl in range(SB):
                pltpu.async_copy(
                    table_hbm.at[idx_v.at[b, sl]], rows_v.at[b, sl], gsems[b]
                )

        def store_waits(g, b):
            for sl in range(SB):
                for ti in range(DT):
                    pltpu.make_async_copy(
                        x_v.at[b, sl, ti], out_hbm.at[g * SB + sl, ti, wid],
                        ssems[b],
                    ).wait()

        def process(g, b):
            @pl.when(g >= 2)
            def _():
                store_waits(g - 2, b)

            pltpu.sync_copy(posx_hbm.at[pl.ds(g * SB, SB)], x_v.at[b])

            for sl in range(SB):
                pltpu.make_async_copy(
                    table_hbm.at[idx_v.at[b, sl]], rows_v.at[b, sl], gsems[b]
                ).wait()

            lanes = lax.iota(jnp.int32, L)

            @plsc.parallel_loop(0, CC * SB, 1, unroll=4)
            def _(i):
                cc = i // SB
                sl = i - cc * SB
                rvec = lanes + cc * L
                for d in range(D):
                    v = plsc.load_gather(
                        rows_v.at[b, sl], [rvec, jnp.full((L,), d, jnp.int32)]
                    )
                    plsc.addupdate(
                        x_v.at[b, sl, d // 8, d % 8, pl.ds(cc * L, L)], v
                    )

            for sl in range(SB):
                for ti in range(DT):
                    pltpu.async_copy(
                        x_v.at[b, sl, ti], out_hbm.at[g * SB + sl, ti, wid],
                        ssems[b],
                    )

        fire(0, 0)

        def body(gg, carry):
            g0 = 2 * gg
            fire(g0 + 1, 1)
            process(g0, 0)

            @pl.when(g0 + 2 < STEPS)
            def _():
                fire(g0 + 2, 0)

            process(g0 + 1, 1)
            return carry

        lax.fori_loop(0, STEPS // 2, body, 0)
        store_waits(STEPS - 2, 0)
        store_waits(STEPS - 1, 1)

    return k


def kernel(inputs, word_table, pos_embed):
    B, S = inputs.shape
    V, D = word_table.shape
    idx_t = inputs.astype(jnp.int32).T
    pos_x = jnp.broadcast_to(
        pos_embed.reshape(S, D // 8, 8)[:, :, :, None], (S, D // 8, 8, BT)
    )
    table_p = jnp.pad(word_table, ((0, 0), (0, PAD - D)))
    x = _make_kernel(B, S, V, D)(idx_t, table_p, pos_x)
    return x.transpose(2, 4, 0, 1, 3).reshape(B, S, D)


# speedup vs baseline: 2.5872x; 1.1507x over previous
import functools
import jax
import jax.numpy as jnp
from jax import lax
from jax.experimental import pallas as pl
from jax.experimental.pallas import tpu as pltpu
from jax.experimental.pallas import tpu_sc as plsc

NC = 2
NS = 16
NW = NC * NS

BT = 128
XP = BT + 1
SB = 4
L = 16


def _make_kernel(B, S, V, D):
    assert B == BT * NW and S % (2 * SB) == 0 and D % 8 == 0
    STEPS = S // SB
    DT = D // 8

    mesh = plsc.VectorSubcoreMesh(core_axis_name="c", subcore_axis_name="s")

    @functools.partial(
        pl.kernel,
        mesh=mesh,
        out_type=jax.ShapeDtypeStruct((S, DT, NW, 8, BT), jnp.float32),
        scratch_types=[
            pltpu.VMEM((2, SB, BT), jnp.int32),
            pltpu.VMEM((2, SB, BT, D), jnp.float32),
            pltpu.VMEM((2, SB, D, XP), jnp.float32),
            pltpu.VMEM((2, SB, D), jnp.float32),
            pltpu.SemaphoreType.DMA,
            pltpu.SemaphoreType.DMA,
            pltpu.SemaphoreType.DMA,
            pltpu.SemaphoreType.DMA,
        ],
        compiler_params=pltpu.CompilerParams(
            use_tc_tiling_on_sc=False, needs_layout_passes=False
        ),
    )
    def k(idx_hbm, table_hbm, pos_hbm, out_hbm, idx_v, rows_v, x_v, pos_v,
          gsem0, gsem1, ssem0, ssem1):
        wid = lax.axis_index("s") * NC + lax.axis_index("c")
        bbase = wid * BT
        gsems = (gsem0, gsem1)
        ssems = (ssem0, ssem1)

        def fire(g, b):
            s0 = g * SB
            pltpu.sync_copy(
                idx_hbm.at[pl.ds(s0, SB), pl.ds(bbase, BT)], idx_v.at[b]
            )
            pltpu.sync_copy(pos_hbm.at[pl.ds(s0, SB)], pos_v.at[b])
            for sl in range(SB):
                pltpu.async_copy(
                    table_hbm.at[idx_v.at[b, sl]], rows_v.at[b, sl], gsems[b]
                )

        def store_waits(g, b):
            for sl in range(SB):
                for ti in range(DT):
                    pltpu.make_async_copy(
                        x_v.at[b, sl, pl.ds(ti * 8, 8), pl.ds(0, BT)],
                        out_hbm.at[g * SB + sl, ti, wid],
                        ssems[b],
                    ).wait()

        def process(g, b):
            @pl.when(g >= 2)
            def _():
                store_waits(g - 2, b)

            for sl in range(SB):
                pltpu.make_async_copy(
                    table_hbm.at[idx_v.at[b, sl]], rows_v.at[b, sl], gsems[b]
                ).wait()

            dlo = lax.iota(jnp.int32, L)
            dhi = dlo + L

            @plsc.parallel_loop(0, BT * SB, 1, unroll=4)
            def _(i):
                c = i // SB
                sl = i - c * SB
                cvec = jnp.full((L,), c, jnp.int32)
                v_lo = rows_v[b, sl, c, pl.ds(0, L)] + pos_v[b, sl, pl.ds(0, L)]
                v_hi = rows_v[b, sl, c, pl.ds(L, L)] + pos_v[b, sl, pl.ds(L, L)]
                plsc.store_scatter(x_v.at[b, sl], [dlo, cvec], v_lo)
                plsc.store_scatter(x_v.at[b, sl], [dhi, cvec], v_hi)

            for sl in range(SB):
                for ti in range(DT):
                    pltpu.async_copy(
                        x_v.at[b, sl, pl.ds(ti * 8, 8), pl.ds(0, BT)],
                        out_hbm.at[g * SB + sl, ti, wid],
                        ssems[b],
                    )

        fire(0, 0)

        def body(gg, carry):
            g0 = 2 * gg
            fire(g0 + 1, 1)
            process(g0, 0)

            @pl.when(g0 + 2 < STEPS)
            def _():
                fire(g0 + 2, 0)

            process(g0 + 1, 1)
            return carry

        lax.fori_loop(0, STEPS // 2, body, 0)
        store_waits(STEPS - 2, 0)
        store_waits(STEPS - 1, 1)

    return k


def kernel(inputs, word_table, pos_embed):
    B, S = inputs.shape
    V, D = word_table.shape
    idx_t = inputs.astype(jnp.int32).T
    x = _make_kernel(B, S, V, D)(idx_t, word_table, pos_embed)
    return x.transpose(2, 4, 0, 1, 3).reshape(B, S, D)
